# baseline (speedup 1.0000x reference)
import functools

import jax
import jax.numpy as jnp
from jax import lax
from jax.experimental import pallas as pl
from jax.experimental.pallas import tpu as pltpu
from jax.experimental.pallas import tpu_sc as plsc

_B, _S, _V = 32, 32, 32768
_L = 16
_NSLICE = _V // _L
_NEG = -3.0e38

_TCB = 20
_NW = 32
_T = (_B - _TCB) * _S // _NW

_KS = 32
_CH = _NSLICE // _KS


def _row_top2(row_ref):
    def chunk_step(c, carry):
        cm1, ci1, cm2 = carry
        base = c * (_KS * _L)
        mchunk = row_ref[pl.ds(base, _L)]
        for k in range(1, _KS):
            mchunk = jnp.maximum(mchunk, row_ref[pl.ds(base + k * _L, _L)])
        cidv = jnp.full((_L,), c, jnp.int32)
        cgt = mchunk > cm1
        cm2 = jnp.where(cgt, cm1, jnp.maximum(cm2, mchunk))
        ci1 = jnp.where(cgt, cidv, ci1)
        cm1 = jnp.maximum(cm1, mchunk)
        return cm1, ci1, cm2

    cinit = (
        jnp.full((_L,), _NEG, jnp.float32),
        jnp.zeros((_L,), jnp.int32),
        jnp.full((_L,), _NEG, jnp.float32),
    )
    cm1, ci1, cm2 = lax.fori_loop(0, _CH, chunk_step, cinit)

    gm1 = jnp.max(cm1)
    ceq = cm1 == gm1
    ci1f = ci1.astype(jnp.float32)
    cstarf = jnp.min(jnp.where(ceq, ci1f, float(_CH)))
    lane_star = ceq & (ci1f == cstarf)
    cross_m2 = jnp.max(jnp.where(lane_star, cm2, cm1))
    cstar = cstarf.astype(jnp.int32)

    rbase = cstar * (_KS * _L)

    def rescan_step(k, carry):
        m1, i1, m2, idx = carry
        v = row_ref[pl.ds(rbase + k * _L, _L)]
        c1 = v > m1
        m2 = jnp.where(c1, m1, jnp.maximum(m2, v))
        i1 = jnp.where(c1, idx, i1)
        m1 = jnp.maximum(m1, v)
        return m1, i1, m2, idx + _L

    rinit = (
        jnp.full((_L,), _NEG, jnp.float32),
        jnp.zeros((_L,), jnp.int32),
        jnp.full((_L,), _NEG, jnp.float32),
        rbase + lax.iota(jnp.int32, _L),
    )
    m1r, i1r, m2r, _ = lax.fori_loop(0, _KS, rescan_step, rinit, unroll=8)

    req = m1r == gm1
    i1f = i1r.astype(jnp.float32)
    gi1f = jnp.min(jnp.where(req, i1f, float(_V)))
    lane_star2 = req & (i1f == gi1f)
    in_m2 = jnp.max(jnp.where(lane_star2, m2r, m1r))
    gm2 = jnp.maximum(cross_m2, in_m2)
    return gm1, gi1f.astype(jnp.int32), gm2


_NBUF = 3


def _sc_body(x_hbm, gold_hbm, out_hbm, row_a, row_b, row_c, gold_v, res_v,
             sem_a, sem_b, sem_c):
    cid = lax.axis_index("c")
    sid = lax.axis_index("s")
    wid = sid * 2 + cid

    base = _TCB * _S + wid * _T
    abase = jnp.minimum((base // 8) * 8, _B * _S - 2 * _L)
    off = base - abase
    pltpu.sync_copy(gold_hbm.at[pl.ds(abase, 2 * _L)], gold_v)
    g0f = gold_v[pl.ds(0, _L)].astype(jnp.float32)
    g1f = gold_v[pl.ds(_L, _L)].astype(jnp.float32)
    lane_iota = lax.iota(jnp.int32, _L)

    rows = (row_a, row_b, row_c)
    sems = (sem_a, sem_b, sem_c)
    cps = [
        pltpu.async_copy(x_hbm.at[base + i], rows[i], sems[i])
        for i in range(_NBUF - 1)
    ] + [None]
    first_b = base // _S
    acc = [jnp.float32(0.0), jnp.float32(0.0)]
    any_nz = [jnp.float32(0.0), jnp.float32(0.0)]
    for s in range(_T):
        if s + _NBUF - 1 < _T:
            cps[(s + _NBUF - 1) % _NBUF] = pltpu.async_copy(
                x_hbm.at[base + s + _NBUF - 1],
                rows[(s + _NBUF - 1) % _NBUF],
                sems[(s + _NBUF - 1) % _NBUF],
            )
        cps[s % _NBUF].wait()
        row_ref = rows[s % _NBUF]

        lane = off + s
        y = jnp.maximum(
            jnp.max(jnp.where(lane_iota == lane, g0f, -1.0)),
            jnp.max(jnp.where(lane_iota == lane - _L, g1f, -1.0)),
        ).astype(jnp.int32)
        y_vec = jnp.full((_L,), y, jnp.int32)
        m1, i1, m2 = _row_top2(row_ref)
        g_vec = plsc.load_gather(row_ref, [y_vec])
        g = jnp.max(g_vec)

        neg = jnp.where(i1 == y, m2, m1)
        lossrow = jnp.maximum(1.0 + neg - g, 0.0)
        lossrow = jnp.where(y != 0, lossrow, 0.0)
        in1 = ((base + s) // _S) != first_b
        nz = jnp.where(y != 0, 1.0, 0.0)
        l0 = jnp.where(in1, 0.0, lossrow)
        n0 = jnp.where(in1, 0.0, nz)
        acc[0] = acc[0] + l0
        acc[1] = acc[1] + (lossrow - l0)
        any_nz[0] = any_nz[0] + n0
        any_nz[1] = any_nz[1] + (nz - n0)

    res_v[0] = jnp.full((_L,), acc[0], jnp.float32)
    res_v[1] = jnp.full((_L,), any_nz[0], jnp.float32)
    res_v[2] = jnp.full((_L,), acc[1], jnp.float32)
    res_v[3] = jnp.full((_L,), any_nz[1], jnp.float32)
    pltpu.sync_copy(res_v, out_hbm.at[wid])


_CW = 256
_NCH = _V // _CW


def _tc_body(x_ref, gold_ref, out_ref):
    b = pl.program_id(0)
    xb = x_ref[0]
    y = gold_ref[0, 0]
    iota = lax.broadcasted_iota(jnp.int32, (_S, _V), 1)
    m1 = jnp.max(xb, axis=1)
    i1 = jnp.min(jnp.where(xb == m1[:, None], iota, _V), axis=1)
    m2 = jnp.max(jnp.where(iota == i1[:, None], _NEG, xb), axis=1)
    g = jnp.max(jnp.where(iota == y[:, None], xb, _NEG), axis=1)

    neg = jnp.where(i1 == y, m2, m1)
    loss = jnp.maximum(1.0 + neg - g, 0.0)
    loss = jnp.where(y != 0, loss, 0.0)
    ltot = jnp.sum(loss)
    anynz = jnp.any(y != 0)
    contrib = (jnp.where(anynz, ltot, 0.0) * (1.0 / _B))[None, None]

    @pl.when(b == 0)
    def _():
        out_ref[:, :] = contrib

    @pl.when(b > 0)
    def _():
        out_ref[:, :] = out_ref[:, :] + contrib


@jax.jit
def kernel(x, gold):
    x2 = x.reshape(_B * _S, _V)
    gold2 = gold.astype(jnp.int32).reshape(_B, _S)
    goldflat = gold2.reshape(_B * _S)

    sc_run = functools.partial(
        pl.kernel,
        mesh=plsc.VectorSubcoreMesh(core_axis_name="c", subcore_axis_name="s"),
        out_type=jax.ShapeDtypeStruct((_NW, 4, _L), jnp.float32),
        scratch_types=[
            pltpu.VMEM((_V,), jnp.float32),
            pltpu.VMEM((_V,), jnp.float32),
            pltpu.VMEM((_V,), jnp.float32),
            pltpu.VMEM((2 * _L,), jnp.int32),
            pltpu.VMEM((4, _L), jnp.float32),
            pltpu.SemaphoreType.DMA,
            pltpu.SemaphoreType.DMA,
            pltpu.SemaphoreType.DMA,
        ],
        compiler_params=pltpu.CompilerParams(needs_layout_passes=False),
    )(_sc_body)
    sc_out = sc_run(x2, goldflat)

    tc_out = pl.pallas_call(
        _tc_body,
        grid=(_TCB,),
        in_specs=[
            pl.BlockSpec((1, _S, _V), lambda b: (b, 0, 0)),
            pl.BlockSpec((1, 1, _S), lambda b: (b, 0, 0)),
        ],
        out_specs=pl.BlockSpec((1, 1), lambda b: (0, 0)),
        out_shape=jax.ShapeDtypeStruct((1, 1), jnp.float32),
    )(x, gold2.reshape(_B, 1, _S))

    seg_batch = []
    for w in range(_NW):
        b0 = (_TCB * _S + w * _T) // _S
        seg_batch += [b0, min(b0 + 1, _B - 1)]
    seg_ids = jnp.asarray(seg_batch, dtype=jnp.int32)
    psums = sc_out[:, (0, 2), 0].reshape(-1)
    panys = sc_out[:, (1, 3), 0].reshape(-1)
    bsum = jax.ops.segment_sum(psums, seg_ids, num_segments=_B)
    bany = jax.ops.segment_sum(panys, seg_ids, num_segments=_B)
    sc_part = jnp.sum(jnp.where(bany > 0, bsum, 0.0)) * (1.0 / _B)
    return tc_out[0, 0] + sc_part


# speedup vs baseline: 1.0442x; 1.0442x over previous
import functools

import jax
import jax.numpy as jnp
from jax import lax
from jax.experimental import pallas as pl
from jax.experimental.pallas import tpu as pltpu
from jax.experimental.pallas import tpu_sc as plsc

_B, _S, _V = 32, 32, 32768
_L = 16
_NSLICE = _V // _L
_NEG = -3.0e38

_TCB = 16
_NW = 32
_T = (_B - _TCB) * _S // _NW

_KS = 32
_CH = _NSLICE // _KS


def _row_top2(row_ref):
    def chunk_step(c, carry):
        cm1, ci1, cm2 = carry
        base = c * (_KS * _L)
        mchunk = row_ref[pl.ds(base, _L)]
        for k in range(1, _KS):
            mchunk = jnp.maximum(mchunk, row_ref[pl.ds(base + k * _L, _L)])
        cidv = jnp.full((_L,), c, jnp.int32)
        cgt = mchunk > cm1
        cm2 = jnp.where(cgt, cm1, jnp.maximum(cm2, mchunk))
        ci1 = jnp.where(cgt, cidv, ci1)
        cm1 = jnp.maximum(cm1, mchunk)
        return cm1, ci1, cm2

    cinit = (
        jnp.full((_L,), _NEG, jnp.float32),
        jnp.zeros((_L,), jnp.int32),
        jnp.full((_L,), _NEG, jnp.float32),
    )
    cm1, ci1, cm2 = lax.fori_loop(0, _CH, chunk_step, cinit)

    gm1 = jnp.max(cm1)
    ceq = cm1 == gm1
    ci1f = ci1.astype(jnp.float32)
    cstarf = jnp.min(jnp.where(ceq, ci1f, float(_CH)))
    lane_star = ceq & (ci1f == cstarf)
    cross_m2 = jnp.max(jnp.where(lane_star, cm2, cm1))
    cstar = cstarf.astype(jnp.int32)

    rbase = cstar * (_KS * _L)

    def rescan_step(k, carry):
        m1, i1, m2, idx = carry
        v = row_ref[pl.ds(rbase + k * _L, _L)]
        c1 = v > m1
        m2 = jnp.where(c1, m1, jnp.maximum(m2, v))
        i1 = jnp.where(c1, idx, i1)
        m1 = jnp.maximum(m1, v)
        return m1, i1, m2, idx + _L

    rinit = (
        jnp.full((_L,), _NEG, jnp.float32),
        jnp.zeros((_L,), jnp.int32),
        jnp.full((_L,), _NEG, jnp.float32),
        rbase + lax.iota(jnp.int32, _L),
    )
    m1r, i1r, m2r, _ = lax.fori_loop(0, _KS, rescan_step, rinit, unroll=8)

    req = m1r == gm1
    i1f = i1r.astype(jnp.float32)
    gi1f = jnp.min(jnp.where(req, i1f, float(_V)))
    lane_star2 = req & (i1f == gi1f)
    in_m2 = jnp.max(jnp.where(lane_star2, m2r, m1r))
    gm2 = jnp.maximum(cross_m2, in_m2)
    return gm1, gi1f.astype(jnp.int32), gm2


_NBUF = 3


def _sc_body(x_hbm, gold_hbm, out_hbm, row_a, row_b, row_c, gold_v, res_v,
             sem_a, sem_b, sem_c):
    cid = lax.axis_index("c")
    sid = lax.axis_index("s")
    wid = sid * 2 + cid

    base = _TCB * _S + wid * _T
    abase = jnp.minimum((base // 8) * 8, _B * _S - 2 * _L)
    off = base - abase
    pltpu.sync_copy(gold_hbm.at[pl.ds(abase, 2 * _L)], gold_v)
    g0f = gold_v[pl.ds(0, _L)].astype(jnp.float32)
    g1f = gold_v[pl.ds(_L, _L)].astype(jnp.float32)
    lane_iota = lax.iota(jnp.int32, _L)

    rows = (row_a, row_b, row_c)
    sems = (sem_a, sem_b, sem_c)
    cps = [
        pltpu.async_copy(x_hbm.at[base + i], rows[i], sems[i])
        for i in range(_NBUF - 1)
    ] + [None]
    first_b = base // _S
    acc = [jnp.float32(0.0), jnp.float32(0.0)]
    any_nz = [jnp.float32(0.0), jnp.float32(0.0)]
    for s in range(_T):
        if s + _NBUF - 1 < _T:
            cps[(s + _NBUF - 1) % _NBUF] = pltpu.async_copy(
                x_hbm.at[base + s + _NBUF - 1],
                rows[(s + _NBUF - 1) % _NBUF],
                sems[(s + _NBUF - 1) % _NBUF],
            )
        cps[s % _NBUF].wait()
        row_ref = rows[s % _NBUF]

        lane = off + s
        y = jnp.maximum(
            jnp.max(jnp.where(lane_iota == lane, g0f, -1.0)),
            jnp.max(jnp.where(lane_iota == lane - _L, g1f, -1.0)),
        ).astype(jnp.int32)
        y_vec = jnp.full((_L,), y, jnp.int32)
        m1, i1, m2 = _row_top2(row_ref)
        g_vec = plsc.load_gather(row_ref, [y_vec])
        g = jnp.max(g_vec)

        neg = jnp.where(i1 == y, m2, m1)
        lossrow = jnp.maximum(1.0 + neg - g, 0.0)
        lossrow = jnp.where(y != 0, lossrow, 0.0)
        in1 = ((base + s) // _S) != first_b
        nz = jnp.where(y != 0, 1.0, 0.0)
        l0 = jnp.where(in1, 0.0, lossrow)
        n0 = jnp.where(in1, 0.0, nz)
        acc[0] = acc[0] + l0
        acc[1] = acc[1] + (lossrow - l0)
        any_nz[0] = any_nz[0] + n0
        any_nz[1] = any_nz[1] + (nz - n0)

    res_v[0] = jnp.full((_L,), acc[0], jnp.float32)
    res_v[1] = jnp.full((_L,), any_nz[0], jnp.float32)
    res_v[2] = jnp.full((_L,), acc[1], jnp.float32)
    res_v[3] = jnp.full((_L,), any_nz[1], jnp.float32)
    pltpu.sync_copy(res_v, out_hbm.at[wid])


_CW = 256
_NCH = _V // _CW


def _tc_body(x_ref, gold_ref, out_ref):
    b = pl.program_id(0)
    xb = x_ref[0]
    y = gold_ref[0, 0]
    iota = lax.broadcasted_iota(jnp.int32, (_S, _V), 1)
    m1 = jnp.max(xb, axis=1)
    i1 = jnp.min(jnp.where(xb == m1[:, None], iota, _V), axis=1)
    m2 = jnp.max(jnp.where(iota == i1[:, None], _NEG, xb), axis=1)
    g = jnp.max(jnp.where(iota == y[:, None], xb, _NEG), axis=1)

    neg = jnp.where(i1 == y, m2, m1)
    loss = jnp.maximum(1.0 + neg - g, 0.0)
    loss = jnp.where(y != 0, loss, 0.0)
    ltot = jnp.sum(loss)
    anynz = jnp.any(y != 0)
    contrib = (jnp.where(anynz, ltot, 0.0) * (1.0 / _B))[None, None]

    @pl.when(b == 0)
    def _():
        out_ref[:, :] = contrib

    @pl.when(b > 0)
    def _():
        out_ref[:, :] = out_ref[:, :] + contrib


@jax.jit
def kernel(x, gold):
    x2 = x.reshape(_B * _S, _V)
    gold2 = gold.astype(jnp.int32).reshape(_B, _S)
    goldflat = gold2.reshape(_B * _S)

    sc_run = functools.partial(
        pl.kernel,
        mesh=plsc.VectorSubcoreMesh(core_axis_name="c", subcore_axis_name="s"),
        out_type=jax.ShapeDtypeStruct((_NW, 4, _L), jnp.float32),
        scratch_types=[
            pltpu.VMEM((_V,), jnp.float32),
            pltpu.VMEM((_V,), jnp.float32),
            pltpu.VMEM((_V,), jnp.float32),
            pltpu.VMEM((2 * _L,), jnp.int32),
            pltpu.VMEM((4, _L), jnp.float32),
            pltpu.SemaphoreType.DMA,
            pltpu.SemaphoreType.DMA,
            pltpu.SemaphoreType.DMA,
        ],
        compiler_params=pltpu.CompilerParams(needs_layout_passes=False),
    )(_sc_body)
    sc_out = sc_run(x2, goldflat)

    tc_out = pl.pallas_call(
        _tc_body,
        grid=(_TCB,),
        in_specs=[
            pl.BlockSpec((1, _S, _V), lambda b: (b, 0, 0)),
            pl.BlockSpec((1, 1, _S), lambda b: (b, 0, 0)),
        ],
        out_specs=pl.BlockSpec((1, 1), lambda b: (0, 0)),
        out_shape=jax.ShapeDtypeStruct((1, 1), jnp.float32),
    )(x, gold2.reshape(_B, 1, _S))

    seg_batch = []
    for w in range(_NW):
        b0 = (_TCB * _S + w * _T) // _S
        seg_batch += [b0, min(b0 + 1, _B - 1)]
    seg_ids = jnp.asarray(seg_batch, dtype=jnp.int32)
    psums = sc_out[:, (0, 2), 0].reshape(-1)
    panys = sc_out[:, (1, 3), 0].reshape(-1)
    bsum = jax.ops.segment_sum(psums, seg_ids, num_segments=_B)
    bany = jax.ops.segment_sum(panys, seg_ids, num_segments=_B)
    sc_part = jnp.sum(jnp.where(bany > 0, bsum, 0.0)) * (1.0 / _B)
    return tc_out[0, 0] + sc_part


# speedup vs baseline: 1.4134x; 1.3535x over previous
import functools

import jax
import jax.numpy as jnp
from jax import lax
from jax.experimental import pallas as pl
from jax.experimental.pallas import tpu as pltpu
from jax.experimental.pallas import tpu_sc as plsc

_B, _S, _V = 32, 32, 32768
_L = 16
_NSLICE = _V // _L
_NEG = -3.0e38

_TCB = 16
_NW = 32
_T = (_B - _TCB) * _S // _NW

_KS = 32
_CH = _NSLICE // _KS


def _row_top2(row_ref):
    def chunk_step(c, carry):
        cm1, ci1, cm2 = carry
        base = c * (_KS * _L)
        mchunk = row_ref[pl.ds(base, _L)]
        for k in range(1, _KS):
            mchunk = jnp.maximum(mchunk, row_ref[pl.ds(base + k * _L, _L)])
        cidv = jnp.full((_L,), c, jnp.int32)
        cgt = mchunk > cm1
        cm2 = jnp.where(cgt, cm1, jnp.maximum(cm2, mchunk))
        ci1 = jnp.where(cgt, cidv, ci1)
        cm1 = jnp.maximum(cm1, mchunk)
        return cm1, ci1, cm2

    cinit = (
        jnp.full((_L,), _NEG, jnp.float32),
        jnp.zeros((_L,), jnp.int32),
        jnp.full((_L,), _NEG, jnp.float32),
    )
    cm1, ci1, cm2 = lax.fori_loop(0, _CH, chunk_step, cinit)

    gm1 = jnp.max(cm1)
    ceq = cm1 == gm1
    ci1f = ci1.astype(jnp.float32)
    cstarf = jnp.min(jnp.where(ceq, ci1f, float(_CH)))
    lane_star = ceq & (ci1f == cstarf)
    cross_m2 = jnp.max(jnp.where(lane_star, cm2, cm1))
    cstar = cstarf.astype(jnp.int32)

    rbase = cstar * (_KS * _L)

    def rescan_step(k, carry):
        m1, i1, m2, idx = carry
        v = row_ref[pl.ds(rbase + k * _L, _L)]
        c1 = v > m1
        m2 = jnp.where(c1, m1, jnp.maximum(m2, v))
        i1 = jnp.where(c1, idx, i1)
        m1 = jnp.maximum(m1, v)
        return m1, i1, m2, idx + _L

    rinit = (
        jnp.full((_L,), _NEG, jnp.float32),
        jnp.zeros((_L,), jnp.int32),
        jnp.full((_L,), _NEG, jnp.float32),
        rbase + lax.iota(jnp.int32, _L),
    )
    m1r, i1r, m2r, _ = lax.fori_loop(0, _KS, rescan_step, rinit, unroll=8)

    req = m1r == gm1
    i1f = i1r.astype(jnp.float32)
    gi1f = jnp.min(jnp.where(req, i1f, float(_V)))
    lane_star2 = req & (i1f == gi1f)
    in_m2 = jnp.max(jnp.where(lane_star2, m2r, m1r))
    gm2 = jnp.maximum(cross_m2, in_m2)
    return gm1, gi1f.astype(jnp.int32), gm2


_NBUF = 3


def _sc_body(x_hbm, gold_hbm, out_hbm, row_a, row_b, row_c, gold_v, res_v,
             sem_a, sem_b, sem_c):
    cid = lax.axis_index("c")
    sid = lax.axis_index("s")
    wid = sid * 2 + cid

    base = _TCB * _S + wid * _T
    abase = jnp.minimum((base // 8) * 8, _B * _S - 2 * _L)
    off = base - abase
    pltpu.sync_copy(gold_hbm.at[pl.ds(abase, 2 * _L)], gold_v)
    g0f = gold_v[pl.ds(0, _L)].astype(jnp.float32)
    g1f = gold_v[pl.ds(_L, _L)].astype(jnp.float32)
    lane_iota = lax.iota(jnp.int32, _L)

    rows = (row_a, row_b, row_c)
    sems = (sem_a, sem_b, sem_c)
    cps = [
        pltpu.async_copy(x_hbm.at[base + i], rows[i], sems[i])
        for i in range(_NBUF - 1)
    ] + [None]
    first_b = base // _S
    acc = [jnp.float32(0.0), jnp.float32(0.0)]
    any_nz = [jnp.float32(0.0), jnp.float32(0.0)]
    for s in range(_T):
        if s + _NBUF - 1 < _T:
            cps[(s + _NBUF - 1) % _NBUF] = pltpu.async_copy(
                x_hbm.at[base + s + _NBUF - 1],
                rows[(s + _NBUF - 1) % _NBUF],
                sems[(s + _NBUF - 1) % _NBUF],
            )
        cps[s % _NBUF].wait()
        row_ref = rows[s % _NBUF]

        lane = off + s
        y = jnp.maximum(
            jnp.max(jnp.where(lane_iota == lane, g0f, -1.0)),
            jnp.max(jnp.where(lane_iota == lane - _L, g1f, -1.0)),
        ).astype(jnp.int32)
        y_vec = jnp.full((_L,), y, jnp.int32)
        m1, i1, m2 = _row_top2(row_ref)
        g_vec = plsc.load_gather(row_ref, [y_vec])
        g = jnp.max(g_vec)

        neg = jnp.where(i1 == y, m2, m1)
        lossrow = jnp.maximum(1.0 + neg - g, 0.0)
        lossrow = jnp.where(y != 0, lossrow, 0.0)
        in1 = ((base + s) // _S) != first_b
        nz = jnp.where(y != 0, 1.0, 0.0)
        l0 = jnp.where(in1, 0.0, lossrow)
        n0 = jnp.where(in1, 0.0, nz)
        acc[0] = acc[0] + l0
        acc[1] = acc[1] + (lossrow - l0)
        any_nz[0] = any_nz[0] + n0
        any_nz[1] = any_nz[1] + (nz - n0)

    res_v[0] = jnp.full((_L,), acc[0], jnp.float32)
    res_v[1] = jnp.full((_L,), any_nz[0], jnp.float32)
    res_v[2] = jnp.full((_L,), acc[1], jnp.float32)
    res_v[3] = jnp.full((_L,), any_nz[1], jnp.float32)
    pltpu.sync_copy(res_v, out_hbm.at[wid])


_CW = 256
_NCH = _V // _CW


def _tc_body(x_ref, gold_ref, out_ref):
    b = pl.program_id(0)
    xb = x_ref[0]
    y = gold_ref[0, 0]
    iota = lax.broadcasted_iota(jnp.int32, (_S, _V), 1)
    m1 = jnp.max(xb, axis=1)
    i1 = jnp.min(jnp.where(xb == m1[:, None], iota, _V), axis=1)
    m2 = jnp.max(jnp.where(iota == i1[:, None], _NEG, xb), axis=1)
    g = jnp.max(jnp.where(iota == y[:, None], xb, _NEG), axis=1)

    neg = jnp.where(i1 == y, m2, m1)
    loss = jnp.maximum(1.0 + neg - g, 0.0)
    loss = jnp.where(y != 0, loss, 0.0)
    ltot = jnp.sum(loss)
    anynz = jnp.any(y != 0)
    contrib = (jnp.where(anynz, ltot, 0.0) * (1.0 / _B))[None, None]

    @pl.when(b == 0)
    def _():
        out_ref[:, :] = contrib

    @pl.when(b > 0)
    def _():
        out_ref[:, :] = out_ref[:, :] + contrib


@jax.jit
def kernel(x, gold):
    x2 = x.reshape(_B * _S, _V)
    gold2 = gold.astype(jnp.int32).reshape(_B, _S)
    goldflat = gold2.reshape(_B * _S)

    sc_run = functools.partial(
        pl.kernel,
        mesh=plsc.VectorSubcoreMesh(core_axis_name="c", subcore_axis_name="s"),
        out_type=jax.ShapeDtypeStruct((_NW, 4, _L), jnp.float32),
        scratch_types=[
            pltpu.VMEM((_V,), jnp.float32),
            pltpu.VMEM((_V,), jnp.float32),
            pltpu.VMEM((_V,), jnp.float32),
            pltpu.VMEM((2 * _L,), jnp.int32),
            pltpu.VMEM((4, _L), jnp.float32),
            pltpu.SemaphoreType.DMA,
            pltpu.SemaphoreType.DMA,
            pltpu.SemaphoreType.DMA,
        ],
        compiler_params=pltpu.CompilerParams(needs_layout_passes=False),
    )(_sc_body)
    sc_out = sc_run(x2, goldflat)

    tc_out = pl.pallas_call(
        _tc_body,
        grid=(_TCB,),
        in_specs=[
            pl.BlockSpec((1, _S, _V), lambda b: (b, 0, 0)),
            pl.BlockSpec((1, 1, _S), lambda b: (b, 0, 0)),
        ],
        out_specs=pl.BlockSpec((1, 1), lambda b: (0, 0)),
        out_shape=jax.ShapeDtypeStruct((1, 1), jnp.float32),
    )(x, gold2.reshape(_B, 1, _S))

    seg_batch = []
    for w in range(_NW):
        b0 = (_TCB * _S + w * _T) // _S
        seg_batch += [b0, min(b0 + 1, _B - 1)]
    import numpy as _np

    onehot = _np.zeros((2 * _NW, _B), _np.float32)
    onehot[_np.arange(2 * _NW), _np.asarray(seg_batch)] = 1.0
    onehot_j = jnp.asarray(onehot)
    psums = sc_out[:, (0, 2), 0].reshape(-1)
    panys = sc_out[:, (1, 3), 0].reshape(-1)
    bsum = psums @ onehot_j
    bany = panys @ onehot_j
    sc_part = jnp.sum(jnp.where(bany > 0, bsum, 0.0)) * (1.0 / _B)
    return tc_out[0, 0] + sc_part
